# baseline (speedup 1.0000x reference)
import jax
import jax.numpy as jnp
from jax.experimental import pallas as pl
from jax.experimental.pallas import tpu as pltpu

_ROWS = 27278
_COLS = 1024
_CH = 512
_NFULL = _ROWS // _CH
_TAIL = _ROWS - _NFULL * _CH
_STEPS = _NFULL + 1
_NBUF = 8


def _copy_full(hbm, buf, sem, chunk, slot):
    return pltpu.make_async_copy(
        hbm.at[pl.ds(chunk * _CH, _CH), :], buf.at[slot], sem.at[slot])


def _copy_tail(hbm, tail_buf, sems, idx):
    return pltpu.make_async_copy(
        hbm.at[pl.ds(_NFULL * _CH, _TAIL), :], tail_buf, sems.at[idx])


def _fold(x):
    return jnp.sum(x.reshape(_CH // 8, 8, _COLS), axis=0)


def _masked_sq_cnt(o, t):
    m = t != -1.0
    return jnp.where(m, (o - t) ** 2, 0.0), m.astype(jnp.float32)


def _masked_mse_body(o_hbm, t_hbm, res_ref,
                     o_bufs, t_bufs, o_tail, t_tail,
                     acc_ref, cnt_ref, o_sems, t_sems, tail_sems):
    i = pl.program_id(0)
    slot = jax.lax.rem(i, _NBUF)

    @pl.when(i == 0)
    def _warmup():
        acc_ref[...] = jnp.zeros_like(acc_ref)
        cnt_ref[...] = jnp.zeros_like(cnt_ref)
        for s in range(_NBUF):
            _copy_full(o_hbm, o_bufs, o_sems, s, s).start()
            _copy_full(t_hbm, t_bufs, t_sems, s, s).start()
        _copy_tail(o_hbm, o_tail, tail_sems, 0).start()
        _copy_tail(t_hbm, t_tail, tail_sems, 1).start()

    @pl.when(i < _NFULL)
    def _step_full():
        _copy_full(o_hbm, o_bufs, o_sems, i, slot).wait()
        _copy_full(t_hbm, t_bufs, t_sems, i, slot).wait()
        sq, c = _masked_sq_cnt(o_bufs[slot], t_bufs[slot])
        acc_ref[...] += _fold(sq)
        cnt_ref[...] += _fold(c)

    nxt = i + _NBUF

    @pl.when(nxt < _NFULL)
    def _prefetch_full():
        _copy_full(o_hbm, o_bufs, o_sems, nxt, slot).start()
        _copy_full(t_hbm, t_bufs, t_sems, nxt, slot).start()

    @pl.when(i == _NFULL)
    def _tail_and_finish():
        _copy_tail(o_hbm, o_tail, tail_sems, 0).wait()
        _copy_tail(t_hbm, t_tail, tail_sems, 1).wait()
        sq, c = _masked_sq_cnt(o_tail[...], t_tail[...])
        loss = jnp.sum(acc_ref[...]) + jnp.sum(sq)
        n_obs = jnp.sum(cnt_ref[...]) + jnp.sum(c)
        res_ref[0, 0] = loss / n_obs


def kernel(output, target):
    res = pl.pallas_call(
        _masked_mse_body,
        grid=(_STEPS,),
        in_specs=[
            pl.BlockSpec(memory_space=pl.ANY),
            pl.BlockSpec(memory_space=pl.ANY),
        ],
        out_specs=pl.BlockSpec(memory_space=pltpu.SMEM),
        out_shape=jax.ShapeDtypeStruct((1, 1), jnp.float32),
        scratch_shapes=[
            pltpu.VMEM((_NBUF, _CH, _COLS), jnp.float32),
            pltpu.VMEM((_NBUF, _CH, _COLS), jnp.float32),
            pltpu.VMEM((_TAIL, _COLS), jnp.float32),
            pltpu.VMEM((_TAIL, _COLS), jnp.float32),
            pltpu.VMEM((8, _COLS), jnp.float32),
            pltpu.VMEM((8, _COLS), jnp.float32),
            pltpu.SemaphoreType.DMA((_NBUF,)),
            pltpu.SemaphoreType.DMA((_NBUF,)),
            pltpu.SemaphoreType.DMA((2,)),
        ],
    )(output.T, target.T)
    return res.reshape(())
